# baseline (speedup 1.0000x reference)
import jax
import jax.numpy as jnp
from jax.experimental import pallas as pl
from jax.experimental.pallas import tpu as pltpu

_N = 10000
_D = 128
_OUT = 128
_BM = 400
_R = 2
_STEPS = _N // _BM


def _sage_kernel(adj_hbm, f_hbm, wt_ref, out_hbm, buf, sems,
                 out_stage, out_sems, f_all_ref, f_sem, f_aug_ref):
    def _copy(step, slot):
        return pltpu.make_async_copy(
            adj_hbm.at[pl.ds(step * _BM, _BM), :],
            buf.at[slot],
            sems.at[slot],
        )

    def _out_copy(step, slot):
        return pltpu.make_async_copy(
            out_stage.at[slot],
            out_hbm.at[pl.ds(step * _BM, _BM), :],
            out_sems.at[slot],
        )

    f_copy = pltpu.make_async_copy(f_hbm, f_all_ref, f_sem)
    _copy(0, 0).start()
    f_copy.start()
    for s in range(1, _R - 1):
        _copy(s, s).start()
    f_copy.wait()

    f_all = f_all_ref[...]
    f_aug_ref[:, :_D] = f_all.astype(jnp.bfloat16)
    lane = jax.lax.broadcasted_iota(jnp.int32, (_N, _D), 1)
    f_aug_ref[:, _D:] = jnp.where(lane == 0, 1.0, 0.0).astype(jnp.bfloat16)
    f_aug = f_aug_ref[...]
    w1 = wt_ref[:_D, :]
    w2 = wt_ref[_D:, :]

    for i in range(_STEPS):
        slot = i % _R
        _copy(i, slot).wait()
        nxt = i + _R - 1
        if nxt < _STEPS:
            _copy(nxt, nxt % _R).start()
        a_bf = buf[slot].astype(jnp.bfloat16)
        r = jnp.dot(a_bf, f_aug,
                    preferred_element_type=jnp.float32)
        rowsum = r[:, _D:_D + 1]
        neigh = r[:, :_D] / (rowsum + 1.0)
        out = jnp.dot(f_all_ref[pl.ds(i * _BM, _BM), :], w1,
                      preferred_element_type=jnp.float32)
        out = out + jnp.dot(neigh, w2,
                            preferred_element_type=jnp.float32)
        oslot = i % 2
        if i >= 2:
            _out_copy(i - 2, oslot).wait()
        out_stage[oslot] = out
        _out_copy(i, oslot).start()

    for i in range(_STEPS - 2, _STEPS):
        _out_copy(i, i % 2).wait()


def kernel(adj, features, W):
    wt = W.T
    return pl.pallas_call(
        _sage_kernel,
        in_specs=[
            pl.BlockSpec(memory_space=pltpu.HBM),
            pl.BlockSpec(memory_space=pltpu.HBM),
            pl.BlockSpec(memory_space=pltpu.VMEM),
        ],
        out_specs=pl.BlockSpec(memory_space=pltpu.HBM),
        out_shape=jax.ShapeDtypeStruct((_N, _OUT), jnp.float32),
        scratch_shapes=[
            pltpu.VMEM((_R, _BM, _N), jnp.float32),
            pltpu.SemaphoreType.DMA((_R,)),
            pltpu.VMEM((2, _BM, _OUT), jnp.float32),
            pltpu.SemaphoreType.DMA((2,)),
            pltpu.VMEM((_N, _D), jnp.float32),
            pltpu.SemaphoreType.DMA,
            pltpu.VMEM((_N, 2 * _D), jnp.bfloat16),
        ],
    )(adj, features, wt)
